# Initial kernel scaffold; baseline (speedup 1.0000x reference)
import jax
import jax.numpy as jnp
from jax.experimental import pallas as pl


def kernel(n_vars, n_clauses, clause_index, literal_index, L_init_scale, C_init_scale, LC_scale, CL_scale, CW1, Cb1, CW2, Cb2, LW1, Lb1, LW2, Lb2, VW1, Vb1, VW2, Vb2):
    raise NotImplementedError("write your pallas kernel here")


# speedup vs baseline: 3.3197x; 3.3197x over previous
import functools

import jax
import jax.numpy as jnp
from jax import lax
from jax.experimental import pallas as pl
from jax.experimental.pallas import tpu as pltpu
from jax.experimental.pallas import tpu_sc as plsc

D = 128
W = 128
NSC = 2
NSUB = 16
CHUNK = 2048
ZROWS = 40

_HIGH = lax.Precision.HIGHEST


def _vmesh():
    return plsc.VectorSubcoreMesh(core_axis_name="c", subcore_axis_name="s")


def _zero_fill(zer):
    zvec = jnp.zeros((16,), jnp.float32)

    @pl.loop(0, zer.shape[0])
    def _(i):
        for j in range(D // 16):
            zer[i, pl.ds(j * 16, 16)] = zvec


def _zero_spmem(zer, acc, start, nrows):
    done = 0
    while done < nrows:
        n = min(ZROWS, nrows - done)
        pltpu.sync_copy(zer.at[pl.ds(0, n)], acc.at[pl.ds(start + done, n)])
        done += n


def _split_rows(n):
    per = (n // NSUB) & ~7
    tail = n - NSUB * per
    assert tail % 8 == 0
    return per, tail


def _make_lc_kernel(n_lits, nchunk, nwin):
    out_rows = nchunk * CHUNK
    acc_rows = CHUNK + 128
    per_zero = acc_rows // NSUB
    per_flush = CHUNK // NSUB
    per_load, tail_load = _split_rows(n_lits)
    mesh = _vmesh()

    @functools.partial(
        pl.kernel, mesh=mesh,
        out_type=jax.ShapeDtypeStruct((out_rows, D), jnp.float32),
        scratch_types=[
            pltpu.VMEM_SHARED((n_lits, D), jnp.float32),
            pltpu.VMEM_SHARED((acc_rows, D), jnp.float32),
            pltpu.VMEM((ZROWS, D), jnp.float32),
            pltpu.VMEM((W,), jnp.int32),
            pltpu.VMEM((W,), jnp.int32),
            pltpu.VMEM((W,), jnp.int32),
            pltpu.VMEM((W, D), jnp.float32),
            pltpu.VMEM((pl.cdiv(2 * nchunk, 16) * 16,), jnp.int32),
        ],
    )
    def lc(l_hbm, ck_hbm, lk_hbm, wb_hbm, out_hbm,
           lsp, acc, zer, ckb, lkb, sid, rows, wb):
        c = lax.axis_index("c")
        s = lax.axis_index("s")
        _zero_fill(zer)
        pltpu.sync_copy(wb_hbm, wb)
        pltpu.sync_copy(l_hbm.at[pl.ds(s * per_load, per_load)],
                        lsp.at[pl.ds(s * per_load, per_load)])
        if tail_load:
            @pl.when(s == 0)
            def _():
                base_t = NSUB * per_load
                pltpu.sync_copy(l_hbm.at[pl.ds(base_t, tail_load)],
                                lsp.at[pl.ds(base_t, tail_load)])
        plsc.subcore_barrier()

        for k in range(nchunk):
            @pl.when(k % NSC == c)
            def _chunk(k=k):
                base = k * CHUNK
                _zero_spmem(zer, acc, s * per_zero, per_zero)
                plsc.subcore_barrier()
                off = 2 * k
                vec = wb[pl.ds((off // 16) * 16, 16)]
                w0 = vec[off % 16]
                w1 = vec[off % 16 + 1]

                @pl.loop(w0 + s, w1, step=NSUB)
                def _win(w):
                    e0 = w * W
                    pltpu.sync_copy(ck_hbm.at[pl.ds(e0, W)], ckb)
                    pltpu.sync_copy(lk_hbm.at[pl.ds(e0, W)], lkb)
                    pltpu.sync_copy(lsp.at[lkb], rows)
                    iot = lax.iota(jnp.int32, 16)
                    for j in range(W // 16):
                        v = ckb[pl.ds(j * 16, 16)] - base
                        ok = (v >= 0) & (v < CHUNK)
                        sid[pl.ds(j * 16, 16)] = jnp.where(ok, v, CHUNK + iot)
                    pltpu.sync_copy(rows, acc.at[sid], add=True)

                plsc.subcore_barrier()
                pltpu.sync_copy(
                    acc.at[pl.ds(s * per_flush, per_flush)],
                    out_hbm.at[pl.ds(base + s * per_flush, per_flush)])
                plsc.subcore_barrier()

    return lc


def _make_cl_kernel(n_lits, nwin):
    acc_rows = pl.cdiv(n_lits + NSUB, 128) * 128
    per_zero = acc_rows // NSUB
    per_flush, tail_flush = _split_rows(n_lits)
    wpc = nwin // NSC
    mesh = _vmesh()

    @functools.partial(
        pl.kernel, mesh=mesh,
        out_type=jax.ShapeDtypeStruct((NSC, n_lits, D), jnp.float32),
        scratch_types=[
            pltpu.VMEM_SHARED((acc_rows, D), jnp.float32),
            pltpu.VMEM((ZROWS, D), jnp.float32),
            pltpu.VMEM((W,), jnp.int32),
            pltpu.VMEM((W,), jnp.int32),
            pltpu.VMEM((W, D), jnp.float32),
        ],
    )
    def cl(c_hbm, ck_hbm, lk_hbm, out_hbm, acc, zer, ckb, lkb, rows):
        c = lax.axis_index("c")
        s = lax.axis_index("s")
        _zero_fill(zer)
        _zero_spmem(zer, acc, s * per_zero, per_zero)
        plsc.subcore_barrier()

        lo = c * wpc + s
        hi = jnp.where(c == NSC - 1, nwin, (c + 1) * wpc)

        @pl.loop(lo, hi, step=NSUB)
        def _win(w):
            e0 = w * W
            pltpu.sync_copy(ck_hbm.at[pl.ds(e0, W)], ckb)
            pltpu.sync_copy(lk_hbm.at[pl.ds(e0, W)], lkb)
            pltpu.sync_copy(c_hbm.at[ckb], rows)
            pltpu.sync_copy(rows, acc.at[lkb], add=True)

        plsc.subcore_barrier()
        pltpu.sync_copy(acc.at[pl.ds(s * per_flush, per_flush)],
                        out_hbm.at[c, pl.ds(s * per_flush, per_flush)])
        if tail_flush:
            @pl.when(s == 0)
            def _():
                base_t = NSUB * per_flush
                pltpu.sync_copy(acc.at[pl.ds(base_t, tail_flush)],
                                out_hbm.at[c, pl.ds(base_t, tail_flush)])

    return cl


def _cmlp(c_arr, msgs, w1a, w1b, b1, w2, b2):
    nc = c_arr.shape[0]
    blk = 2048

    def body(x_ref, m_ref, w1a_ref, w1b_ref, b1_ref, w2_ref, b2_ref, o_ref):
        x = x_ref[...]
        m = m_ref[...]
        h = jnp.maximum(
            jnp.dot(x, w1a_ref[...], precision=_HIGH)
            + jnp.dot(m, w1b_ref[...], precision=_HIGH)
            + b1_ref[...], 0.0)
        o_ref[...] = jnp.dot(h, w2_ref[...], precision=_HIGH) + b2_ref[...] + x

    wspec = pl.BlockSpec((D, D), lambda i: (0, 0))
    bspec = pl.BlockSpec((1, D), lambda i: (0, 0))
    return pl.pallas_call(
        body,
        grid=(pl.cdiv(nc, blk),),
        in_specs=[
            pl.BlockSpec((blk, D), lambda i: (i, 0)),
            pl.BlockSpec((blk, D), lambda i: (i, 0)),
            wspec, wspec, bspec, wspec, bspec,
        ],
        out_specs=pl.BlockSpec((blk, D), lambda i: (i, 0)),
        out_shape=jax.ShapeDtypeStruct((nc, D), jnp.float32),
        compiler_params=pltpu.CompilerParams(
            dimension_semantics=("parallel",)),
    )(c_arr, msgs, w1a, w1b, b1.reshape(1, D), w2, b2.reshape(1, D))


def _lmlp(x_v, mp_v, wa, wb, wc, b1, w2, b2):
    nv = x_v.shape[0]

    def body(x_ref, m_ref, wa_ref, wb_ref, wc_ref, b1_ref, w2_ref, b2_ref,
             o_ref):
        x = x_ref[...]
        xp, xn = x[:, :D], x[:, D:]
        m = m_ref[0] + m_ref[1]
        mp, mn = m[:, :D], m[:, D:]
        b1v = b1_ref[...]
        hp = jnp.maximum(
            jnp.dot(xp, wa_ref[...], precision=_HIGH)
            + jnp.dot(mp, wb_ref[...], precision=_HIGH)
            + jnp.dot(xn, wc_ref[...], precision=_HIGH) + b1v, 0.0)
        hn = jnp.maximum(
            jnp.dot(xn, wa_ref[...], precision=_HIGH)
            + jnp.dot(mn, wb_ref[...], precision=_HIGH)
            + jnp.dot(xp, wc_ref[...], precision=_HIGH) + b1v, 0.0)
        yp = jnp.dot(hp, w2_ref[...], precision=_HIGH) + b2_ref[...] + xp
        yn = jnp.dot(hn, w2_ref[...], precision=_HIGH) + b2_ref[...] + xn
        o_ref[...] = jnp.concatenate([yp, yn], axis=1)

    return pl.pallas_call(
        body,
        out_shape=jax.ShapeDtypeStruct((nv, 2 * D), jnp.float32),
    )(x_v, mp_v, wa, wb, wc, b1.reshape(1, D), w2, b2.reshape(1, D))


def _vmlp(v_arr, w1, b1, w2, b2):
    nv = v_arr.shape[0]

    def body(v_ref, w1_ref, b1_ref, w2_ref, b2_ref, o_ref):
        h = jnp.maximum(
            jnp.dot(v_ref[...], w1_ref[...], precision=_HIGH) + b1_ref[...],
            0.0)
        o_ref[...] = jnp.dot(h, w2_ref[...], precision=_HIGH) + b2_ref[...]

    return pl.pallas_call(
        body,
        out_shape=jax.ShapeDtypeStruct((nv, 1), jnp.float32),
    )(v_arr, w1, b1.reshape(1, D), w2, b2.reshape(1, 1))


def kernel(n_vars, n_clauses, clause_index, literal_index,
           L_init_scale, C_init_scale, LC_scale, CL_scale,
           CW1, Cb1, CW2, Cb2, LW1, Lb1, LW2, Lb2, VW1, Vb1, VW2, Vb2):
    nv = n_vars.shape[0]
    n_lits = 2 * nv
    nc = n_clauses.shape[0]
    num_edges = clause_index.shape[0]
    assert num_edges % W == 0 and n_lits % NSUB == 0
    nwin = num_edges // W
    nchunk = pl.cdiv(nc, CHUNK)

    ck, lk = lax.sort((clause_index, literal_index), num_keys=1)
    chunk_starts = jnp.arange(nchunk + 1, dtype=jnp.int32) * CHUNK
    bounds = jnp.searchsorted(ck, chunk_starts, side="left").astype(jnp.int32)
    wlo = bounds[:-1] // W
    whi = (bounds[1:] + (W - 1)) // W
    wb = jnp.stack([wlo, whi], axis=1).reshape(-1)
    wb_len = pl.cdiv(2 * nchunk, 16) * 16
    wb = jnp.pad(wb, (0, wb_len - 2 * nchunk))

    lc_k = _make_lc_kernel(n_lits, nchunk, nwin)
    cl_k = _make_cl_kernel(n_lits, nwin)

    cw1a, cw1b = CW1[:D], CW1[D:] * LC_scale
    lwa, lwb, lwc = LW1[:D], LW1[D:2 * D] * CL_scale, LW1[2 * D:]

    x_v = jnp.full((nv, 2 * D), L_init_scale, jnp.float32)
    c_arr = jnp.full((nc, D), C_init_scale, jnp.float32)

    for _ in range(3):
        l_tab = x_v.reshape(n_lits, D)
        lc_msgs = lc_k(l_tab, ck, lk, wb)
        c_arr = _cmlp(c_arr, lc_msgs, cw1a, cw1b, Cb1, CW2, Cb2)
        cl_part = cl_k(c_arr, ck, lk)
        mp_v = cl_part.reshape(NSC, nv, 2 * D)
        x_v = _lmlp(x_v, mp_v, lwa, lwb, lwc, Lb1, LW2, Lb2)

    out = _vmlp(x_v, VW1, Vb1, VW2, Vb2)
    return out[:, 0]


# speedup vs baseline: 4.0191x; 1.2107x over previous
import functools

import jax
import jax.numpy as jnp
from jax import lax
from jax.experimental import pallas as pl
from jax.experimental.pallas import tpu as pltpu
from jax.experimental.pallas import tpu_sc as plsc

D = 128
W = 128
NSC = 2
NSUB = 16
CHUNK = 2048
ZROWS = 40


def _vmesh():
    return plsc.VectorSubcoreMesh(core_axis_name="c", subcore_axis_name="s")


def _zero_fill(zer):
    zvec = jnp.zeros((16,), jnp.float32)

    @pl.loop(0, zer.shape[0])
    def _(i):
        for j in range(D // 16):
            zer[i, pl.ds(j * 16, 16)] = zvec


def _zero_spmem(zer, acc, start, nrows):
    done = 0
    while done < nrows:
        n = min(ZROWS, nrows - done)
        pltpu.sync_copy(zer.at[pl.ds(0, n)], acc.at[pl.ds(start + done, n)])
        done += n


def _split_rows(n):
    per = (n // NSUB) & ~7
    tail = n - NSUB * per
    assert tail % 8 == 0
    return per, tail


def _make_lc_kernel(n_lits, nchunk, nwin):
    out_rows = nchunk * CHUNK
    acc_rows = CHUNK + 128
    per_zero = acc_rows // NSUB
    per_flush = CHUNK // NSUB
    per_load, tail_load = _split_rows(n_lits)
    mesh = _vmesh()

    @functools.partial(
        pl.kernel, mesh=mesh,
        out_type=jax.ShapeDtypeStruct((out_rows, D), jnp.float32),
        scratch_types=[
            pltpu.VMEM_SHARED((n_lits, D), jnp.float32),
            pltpu.VMEM_SHARED((acc_rows, D), jnp.float32),
            pltpu.VMEM((ZROWS, D), jnp.float32),
            pltpu.VMEM((W,), jnp.int32),
            pltpu.VMEM((W,), jnp.int32),
            pltpu.VMEM((W,), jnp.int32),
            pltpu.VMEM((W, D), jnp.float32),
            pltpu.VMEM((pl.cdiv(2 * nchunk, 16) * 16,), jnp.int32),
        ],
    )
    def lc(l_hbm, ck_hbm, lk_hbm, wb_hbm, out_hbm,
           lsp, acc, zer, ckb, lkb, sid, rows, wb):
        c = lax.axis_index("c")
        s = lax.axis_index("s")
        _zero_fill(zer)
        pltpu.sync_copy(wb_hbm, wb)
        pltpu.sync_copy(l_hbm.at[pl.ds(s * per_load, per_load)],
                        lsp.at[pl.ds(s * per_load, per_load)])
        if tail_load:
            @pl.when(s == 0)
            def _():
                base_t = NSUB * per_load
                pltpu.sync_copy(l_hbm.at[pl.ds(base_t, tail_load)],
                                lsp.at[pl.ds(base_t, tail_load)])
        plsc.subcore_barrier()

        for k in range(nchunk):
            @pl.when(k % NSC == c)
            def _chunk(k=k):
                base = k * CHUNK
                _zero_spmem(zer, acc, s * per_zero, per_zero)
                plsc.subcore_barrier()
                off = 2 * k
                vec = wb[pl.ds((off // 16) * 16, 16)]
                w0 = vec[off % 16]
                w1 = vec[off % 16 + 1]

                @pl.loop(w0 + s, w1, step=NSUB)
                def _win(w):
                    e0 = w * W
                    pltpu.sync_copy(ck_hbm.at[pl.ds(e0, W)], ckb)
                    pltpu.sync_copy(lk_hbm.at[pl.ds(e0, W)], lkb)
                    pltpu.sync_copy(lsp.at[lkb], rows)
                    iot = lax.iota(jnp.int32, 16)
                    for j in range(W // 16):
                        v = ckb[pl.ds(j * 16, 16)] - base
                        ok = (v >= 0) & (v < CHUNK)
                        sid[pl.ds(j * 16, 16)] = jnp.where(ok, v, CHUNK + iot)
                    pltpu.sync_copy(rows, acc.at[sid], add=True)

                plsc.subcore_barrier()
                pltpu.sync_copy(
                    acc.at[pl.ds(s * per_flush, per_flush)],
                    out_hbm.at[pl.ds(base + s * per_flush, per_flush)])
                plsc.subcore_barrier()

    return lc


def _make_cl_kernel(n_lits, nwin):
    acc_rows = pl.cdiv(n_lits + NSUB, 128) * 128
    per_zero = acc_rows // NSUB
    per_flush, tail_flush = _split_rows(n_lits)
    wpc = nwin // NSC
    mesh = _vmesh()

    @functools.partial(
        pl.kernel, mesh=mesh,
        out_type=jax.ShapeDtypeStruct((NSC, n_lits, D), jnp.float32),
        scratch_types=[
            pltpu.VMEM_SHARED((acc_rows, D), jnp.float32),
            pltpu.VMEM((ZROWS, D), jnp.float32),
            pltpu.VMEM((W,), jnp.int32),
            pltpu.VMEM((W,), jnp.int32),
            pltpu.VMEM((W, D), jnp.float32),
        ],
    )
    def cl(c_hbm, ck_hbm, lk_hbm, out_hbm, acc, zer, ckb, lkb, rows):
        c = lax.axis_index("c")
        s = lax.axis_index("s")
        _zero_fill(zer)
        _zero_spmem(zer, acc, s * per_zero, per_zero)
        plsc.subcore_barrier()

        lo = c * wpc + s
        hi = jnp.where(c == NSC - 1, nwin, (c + 1) * wpc)

        @pl.loop(lo, hi, step=NSUB)
        def _win(w):
            e0 = w * W
            pltpu.sync_copy(ck_hbm.at[pl.ds(e0, W)], ckb)
            pltpu.sync_copy(lk_hbm.at[pl.ds(e0, W)], lkb)
            pltpu.sync_copy(c_hbm.at[ckb], rows)
            pltpu.sync_copy(rows, acc.at[lkb], add=True)

        plsc.subcore_barrier()
        pltpu.sync_copy(acc.at[pl.ds(s * per_flush, per_flush)],
                        out_hbm.at[c, pl.ds(s * per_flush, per_flush)])
        if tail_flush:
            @pl.when(s == 0)
            def _():
                base_t = NSUB * per_flush
                pltpu.sync_copy(acc.at[pl.ds(base_t, tail_flush)],
                                out_hbm.at[c, pl.ds(base_t, tail_flush)])

    return cl


def _cmlp(c_arr, msgs, w1, b1, w2, b2):
    nc = c_arr.shape[0]
    blk = 2048

    def body(x_ref, m_ref, w1_ref, b1_ref, w2_ref, b2_ref, o_ref):
        x = x_ref[...]
        xm = jnp.concatenate([x, m_ref[...]], axis=1)
        h = jnp.maximum(jnp.dot(xm, w1_ref[...]) + b1_ref[...], 0.0)
        o_ref[...] = jnp.dot(h, w2_ref[...]) + b2_ref[...] + x

    bspec = pl.BlockSpec((1, D), lambda i: (0, 0))
    return pl.pallas_call(
        body,
        grid=(pl.cdiv(nc, blk),),
        in_specs=[
            pl.BlockSpec((blk, D), lambda i: (i, 0)),
            pl.BlockSpec((blk, D), lambda i: (i, 0)),
            pl.BlockSpec((2 * D, D), lambda i: (0, 0)),
            bspec,
            pl.BlockSpec((D, D), lambda i: (0, 0)),
            bspec,
        ],
        out_specs=pl.BlockSpec((blk, D), lambda i: (i, 0)),
        out_shape=jax.ShapeDtypeStruct((nc, D), jnp.float32),
        compiler_params=pltpu.CompilerParams(
            dimension_semantics=("parallel",)),
    )(c_arr, msgs, w1, b1.reshape(1, D), w2, b2.reshape(1, D))


def _lmlp(x_v, mp_v, wa, wb, wc, b1, w2, b2):
    nv = x_v.shape[0]

    def body(x_ref, m_ref, wa_ref, wb_ref, wc_ref, b1_ref, w2_ref, b2_ref,
             o_ref):
        x = x_ref[...]
        xp, xn = x[:, :D], x[:, D:]
        m = m_ref[0] + m_ref[1]
        mp, mn = m[:, :D], m[:, D:]
        b1v = b1_ref[...]
        hp = jnp.maximum(
            jnp.dot(xp, wa_ref[...])
            + jnp.dot(mp, wb_ref[...])
            + jnp.dot(xn, wc_ref[...]) + b1v, 0.0)
        hn = jnp.maximum(
            jnp.dot(xn, wa_ref[...])
            + jnp.dot(mn, wb_ref[...])
            + jnp.dot(xp, wc_ref[...]) + b1v, 0.0)
        yp = jnp.dot(hp, w2_ref[...]) + b2_ref[...] + xp
        yn = jnp.dot(hn, w2_ref[...]) + b2_ref[...] + xn
        o_ref[...] = jnp.concatenate([yp, yn], axis=1)

    return pl.pallas_call(
        body,
        out_shape=jax.ShapeDtypeStruct((nv, 2 * D), jnp.float32),
    )(x_v, mp_v, wa, wb, wc, b1.reshape(1, D), w2, b2.reshape(1, D))


def _vmlp(v_arr, w1, b1, w2, b2):
    nv = v_arr.shape[0]

    def body(v_ref, w1_ref, b1_ref, w2_ref, b2_ref, o_ref):
        h = jnp.maximum(
            jnp.dot(v_ref[...], w1_ref[...]) + b1_ref[...],
            0.0)
        o_ref[...] = jnp.dot(h, w2_ref[...]) + b2_ref[...]

    return pl.pallas_call(
        body,
        out_shape=jax.ShapeDtypeStruct((nv, 1), jnp.float32),
    )(v_arr, w1, b1.reshape(1, D), w2, b2.reshape(1, 1))


def kernel(n_vars, n_clauses, clause_index, literal_index,
           L_init_scale, C_init_scale, LC_scale, CL_scale,
           CW1, Cb1, CW2, Cb2, LW1, Lb1, LW2, Lb2, VW1, Vb1, VW2, Vb2):
    nv = n_vars.shape[0]
    n_lits = 2 * nv
    nc = n_clauses.shape[0]
    num_edges = clause_index.shape[0]
    assert num_edges % W == 0 and n_lits % NSUB == 0
    nwin = num_edges // W
    nchunk = pl.cdiv(nc, CHUNK)

    ck, lk = lax.sort((clause_index, literal_index), num_keys=1)
    chunk_starts = jnp.arange(nchunk + 1, dtype=jnp.int32) * CHUNK
    bounds = jnp.searchsorted(ck, chunk_starts, side="left").astype(jnp.int32)
    wlo = bounds[:-1] // W
    whi = (bounds[1:] + (W - 1)) // W
    wb = jnp.stack([wlo, whi], axis=1).reshape(-1)
    wb_len = pl.cdiv(2 * nchunk, 16) * 16
    wb = jnp.pad(wb, (0, wb_len - 2 * nchunk))

    lc_k = _make_lc_kernel(n_lits, nchunk, nwin)
    cl_k = _make_cl_kernel(n_lits, nwin)

    cw1 = jnp.concatenate([CW1[:D], CW1[D:] * LC_scale], axis=0)
    lwa, lwb, lwc = LW1[:D], LW1[D:2 * D] * CL_scale, LW1[2 * D:]

    x_v = jnp.full((nv, 2 * D), L_init_scale, jnp.float32)
    c_arr = jnp.full((nc, D), C_init_scale, jnp.float32)

    for _ in range(3):
        l_tab = x_v.reshape(n_lits, D)
        lc_msgs = lc_k(l_tab, ck, lk, wb)
        c_arr = _cmlp(c_arr, lc_msgs, cw1, Cb1, CW2, Cb2)
        cl_part = cl_k(c_arr, ck, lk)
        mp_v = cl_part.reshape(NSC, nv, 2 * D)
        x_v = _lmlp(x_v, mp_v, lwa, lwb, lwc, Lb1, LW2, Lb2)

    out = _vmlp(x_v, VW1, Vb1, VW2, Vb2)
    return out[:, 0]


# speedup vs baseline: 4.3898x; 1.0922x over previous
import functools

import jax
import jax.numpy as jnp
from jax import lax
from jax.experimental import pallas as pl
from jax.experimental.pallas import tpu as pltpu
from jax.experimental.pallas import tpu_sc as plsc

D = 128
W = 128
W2 = 64
NSC = 2
NSUB = 16
CHUNK = 4096
ZROWS = 40


def _vmesh():
    return plsc.VectorSubcoreMesh(core_axis_name="c", subcore_axis_name="s")


def _zero_fill(zer):
    zvec = jnp.zeros((16,), jnp.float32)

    @pl.loop(0, zer.shape[0])
    def _(i):
        for j in range(D // 16):
            zer[i, pl.ds(j * 16, 16)] = zvec


def _zero_spmem(zer, acc, start, nrows, sem=None):
    done = 0
    handles = []
    while done < nrows:
        n = min(zer.shape[0], nrows - done)
        if sem is None:
            pltpu.sync_copy(zer.at[pl.ds(0, n)], acc.at[pl.ds(start + done, n)])
        else:
            handles.append(pltpu.async_copy(
                zer.at[pl.ds(0, n)], acc.at[pl.ds(start + done, n)], sem))
        done += n
    for h in handles:
        h.wait()


def _split_rows(n):
    per = (n // NSUB) & ~7
    tail = n - NSUB * per
    assert tail % 8 == 0
    return per, tail


def _make_lc_kernel(n_lits, nchunk, nwin):
    out_rows = nchunk * CHUNK
    acc_rows = CHUNK + 128
    per_zero = acc_rows // NSUB
    per_flush = CHUNK // NSUB
    per_load, tail_load = _split_rows(n_lits)
    mesh = _vmesh()

    @functools.partial(
        pl.kernel, mesh=mesh,
        out_type=jax.ShapeDtypeStruct((out_rows, D), jnp.float32),
        scratch_types=[
            pltpu.VMEM_SHARED((n_lits, D), jnp.float32),
            pltpu.VMEM_SHARED((acc_rows, D), jnp.float32),
            pltpu.VMEM((ZROWS, D), jnp.float32),
            pltpu.VMEM((2 * W2,), jnp.int32),
            pltpu.VMEM((2 * W2,), jnp.int32),
            pltpu.VMEM((2, W2), jnp.int32),
            pltpu.VMEM((W2, D), jnp.float32),
            pltpu.VMEM((pl.cdiv(2 * nchunk, 16) * 16,), jnp.int32),
            pltpu.SemaphoreType.DMA,
        ],
    )
    def lc(l_hbm, ck_hbm, lk_hbm, wb_hbm, out_hbm,
           lsp, acc, zer, ckb, lkb, sid, rows, wb, sem):
        c = lax.axis_index("c")
        s = lax.axis_index("s")
        _zero_fill(zer)
        pltpu.sync_copy(wb_hbm, wb)
        pltpu.sync_copy(l_hbm.at[pl.ds(s * per_load, per_load)],
                        lsp.at[pl.ds(s * per_load, per_load)])
        if tail_load:
            @pl.when(s == 0)
            def _():
                base_t = NSUB * per_load
                pltpu.sync_copy(l_hbm.at[pl.ds(base_t, tail_load)],
                                lsp.at[pl.ds(base_t, tail_load)])
        plsc.subcore_barrier()
        iot = lax.iota(jnp.int32, 16)

        def _do_window(b, base):
            for j in range(W2 // 16):
                v = ckb[pl.ds(b * W2 + j * 16, 16)] - base
                ok = (v >= 0) & (v < CHUNK)
                sid[b, pl.ds(j * 16, 16)] = jnp.where(ok, v, CHUNK + iot)
            pltpu.sync_copy(lsp.at[lkb.at[pl.ds(b * W2, W2)]], rows)
            pltpu.sync_copy(rows, acc.at[sid.at[b]], add=True)

        for k in range(nchunk):
            @pl.when(k % NSC == c)
            def _chunk(k=k):
                base = k * CHUNK
                _zero_spmem(zer, acc, s * per_zero, per_zero, sem)
                plsc.subcore_barrier()
                off = 2 * k
                vec = wb[pl.ds((off // 16) * 16, 16)]
                w0 = vec[off % 16]
                w1 = vec[off % 16 + 1]

                @pl.loop(w0 + 2 * s, w1, step=2 * NSUB)
                def _pair(w):
                    e0 = w * W2
                    pltpu.sync_copy(ck_hbm.at[pl.ds(e0, 2 * W2)], ckb)
                    pltpu.sync_copy(lk_hbm.at[pl.ds(e0, 2 * W2)], lkb)
                    _do_window(0, base)

                    @pl.when(w + 1 < w1)
                    def _():
                        _do_window(1, base)

                plsc.subcore_barrier()
                pltpu.sync_copy(
                    acc.at[pl.ds(s * per_flush, per_flush)],
                    out_hbm.at[pl.ds(base + s * per_flush, per_flush)])
                plsc.subcore_barrier()

    return lc


def _make_cl_kernel(n_lits, nwin):
    acc_rows = pl.cdiv(n_lits + NSUB, 128) * 128
    per_zero = acc_rows // NSUB
    per_flush, tail_flush = _split_rows(n_lits)
    nw_total = NSC * NSUB
    per_t = pl.cdiv(nwin, nw_total)
    mesh = _vmesh()

    @functools.partial(
        pl.kernel, mesh=mesh,
        out_type=jax.ShapeDtypeStruct((NSC, n_lits, D), jnp.float32),
        scratch_types=[
            pltpu.VMEM_SHARED((acc_rows, D), jnp.float32),
            pltpu.VMEM((ZROWS, D), jnp.float32),
            pltpu.VMEM((2 * W,), jnp.int32),
            pltpu.VMEM((2 * W,), jnp.int32),
            pltpu.VMEM((2, W), jnp.int32),
            pltpu.VMEM((W, D), jnp.float32),
            pltpu.VMEM((W, D), jnp.float32),
            pltpu.SemaphoreType.DMA,
            pltpu.SemaphoreType.DMA,
        ],
    )
    def cl(c_hbm, ck_hbm, lk_hbm, out_hbm, acc, zer, ckb, lkb, sid,
           r0, r1, sm0, sm1):
        c = lax.axis_index("c")
        s = lax.axis_index("s")
        _zero_fill(zer)
        _zero_spmem(zer, acc, s * per_zero, per_zero, sm0)
        plsc.subcore_barrier()

        wid = s * NSC + c
        lo = wid * per_t
        hi = jnp.minimum(nwin, lo + per_t)

        def _sid_copy(b):
            for j in range(W // 16):
                sid[b, pl.ds(j * 16, 16)] = lkb[pl.ds(b * W + j * 16, 16)]

        @pl.loop(lo, hi, step=2)
        def _pair(w):
            e0 = w * W
            pltpu.sync_copy(ck_hbm.at[pl.ds(e0, 2 * W)], ckb)
            pltpu.sync_copy(lk_hbm.at[pl.ds(e0, 2 * W)], lkb)

            @pl.when(w + 1 < hi)
            def _fast():
                h0 = pltpu.async_copy(c_hbm.at[ckb.at[pl.ds(0, W)]], r0, sm0)
                h1 = pltpu.async_copy(c_hbm.at[ckb.at[pl.ds(W, W)]], r1, sm1)
                _sid_copy(0)
                _sid_copy(1)
                h0.wait()
                pltpu.sync_copy(r0, acc.at[sid.at[0]], add=True)
                h1.wait()
                pltpu.sync_copy(r1, acc.at[sid.at[1]], add=True)

            @pl.when(w + 1 >= hi)
            def _slow():
                _sid_copy(0)
                pltpu.sync_copy(c_hbm.at[ckb.at[pl.ds(0, W)]], r0)
                pltpu.sync_copy(r0, acc.at[sid.at[0]], add=True)

        plsc.subcore_barrier()
        pltpu.sync_copy(acc.at[pl.ds(s * per_flush, per_flush)],
                        out_hbm.at[c, pl.ds(s * per_flush, per_flush)])
        if tail_flush:
            @pl.when(s == 0)
            def _():
                base_t = NSUB * per_flush
                pltpu.sync_copy(acc.at[pl.ds(base_t, tail_flush)],
                                out_hbm.at[c, pl.ds(base_t, tail_flush)])

    return cl


def _cmlp(c_arr, msgs, w1, b1, w2, b2):
    nc = c_arr.shape[0]
    blk = 2048

    def body(x_ref, m_ref, w1_ref, b1_ref, w2_ref, b2_ref, o_ref):
        x = x_ref[...]
        xm = jnp.concatenate([x, m_ref[...]], axis=1)
        h = jnp.maximum(jnp.dot(xm, w1_ref[...]) + b1_ref[...], 0.0)
        o_ref[...] = jnp.dot(h, w2_ref[...]) + b2_ref[...] + x

    bspec = pl.BlockSpec((1, D), lambda i: (0, 0))
    return pl.pallas_call(
        body,
        grid=(pl.cdiv(nc, blk),),
        in_specs=[
            pl.BlockSpec((blk, D), lambda i: (i, 0)),
            pl.BlockSpec((blk, D), lambda i: (i, 0)),
            pl.BlockSpec((2 * D, D), lambda i: (0, 0)),
            bspec,
            pl.BlockSpec((D, D), lambda i: (0, 0)),
            bspec,
        ],
        out_specs=pl.BlockSpec((blk, D), lambda i: (i, 0)),
        out_shape=jax.ShapeDtypeStruct((nc, D), jnp.float32),
        compiler_params=pltpu.CompilerParams(
            dimension_semantics=("parallel",)),
    )(c_arr, msgs, w1, b1.reshape(1, D), w2, b2.reshape(1, D))


def _lmlp(x_v, mp_v, wa, wb, wc, b1, w2, b2):
    nv = x_v.shape[0]

    def body(x_ref, m_ref, wa_ref, wb_ref, wc_ref, b1_ref, w2_ref, b2_ref,
             o_ref):
        x = x_ref[...]
        xp, xn = x[:, :D], x[:, D:]
        m = m_ref[0] + m_ref[1]
        mp, mn = m[:, :D], m[:, D:]
        b1v = b1_ref[...]
        hp = jnp.maximum(
            jnp.dot(xp, wa_ref[...])
            + jnp.dot(mp, wb_ref[...])
            + jnp.dot(xn, wc_ref[...]) + b1v, 0.0)
        hn = jnp.maximum(
            jnp.dot(xn, wa_ref[...])
            + jnp.dot(mn, wb_ref[...])
            + jnp.dot(xp, wc_ref[...]) + b1v, 0.0)
        yp = jnp.dot(hp, w2_ref[...]) + b2_ref[...] + xp
        yn = jnp.dot(hn, w2_ref[...]) + b2_ref[...] + xn
        o_ref[...] = jnp.concatenate([yp, yn], axis=1)

    return pl.pallas_call(
        body,
        out_shape=jax.ShapeDtypeStruct((nv, 2 * D), jnp.float32),
    )(x_v, mp_v, wa, wb, wc, b1.reshape(1, D), w2, b2.reshape(1, D))


def _vmlp(v_arr, w1, b1, w2, b2):
    nv = v_arr.shape[0]

    def body(v_ref, w1_ref, b1_ref, w2_ref, b2_ref, o_ref):
        h = jnp.maximum(
            jnp.dot(v_ref[...], w1_ref[...]) + b1_ref[...],
            0.0)
        o_ref[...] = jnp.dot(h, w2_ref[...]) + b2_ref[...]

    return pl.pallas_call(
        body,
        out_shape=jax.ShapeDtypeStruct((nv, 1), jnp.float32),
    )(v_arr, w1, b1.reshape(1, D), w2, b2.reshape(1, 1))


def kernel(n_vars, n_clauses, clause_index, literal_index,
           L_init_scale, C_init_scale, LC_scale, CL_scale,
           CW1, Cb1, CW2, Cb2, LW1, Lb1, LW2, Lb2, VW1, Vb1, VW2, Vb2):
    nv = n_vars.shape[0]
    n_lits = 2 * nv
    nc = n_clauses.shape[0]
    num_edges = clause_index.shape[0]
    assert num_edges % W == 0 and num_edges % W2 == 0 and n_lits % NSUB == 0
    nwin = num_edges // W
    nchunk = pl.cdiv(nc, CHUNK)

    ck, lk = lax.sort((clause_index, literal_index), num_keys=1)
    chunk_starts = jnp.arange(nchunk + 1, dtype=jnp.int32) * CHUNK
    bounds = jnp.searchsorted(ck, chunk_starts, side="left").astype(jnp.int32)
    wlo = bounds[:-1] // W2
    whi = (bounds[1:] + (W2 - 1)) // W2
    wb = jnp.stack([wlo, whi], axis=1).reshape(-1)
    wb_len = pl.cdiv(2 * nchunk, 16) * 16
    wb = jnp.pad(wb, (0, wb_len - 2 * nchunk))
    ck = jnp.pad(ck, (0, W))
    lk = jnp.pad(lk, (0, W))

    lc_k = _make_lc_kernel(n_lits, nchunk, nwin)
    cl_k = _make_cl_kernel(n_lits, nwin)

    cw1 = jnp.concatenate([CW1[:D], CW1[D:] * LC_scale], axis=0)
    lwa, lwb, lwc = LW1[:D], LW1[D:2 * D] * CL_scale, LW1[2 * D:]

    x_v = jnp.full((nv, 2 * D), L_init_scale, jnp.float32)
    c_arr = jnp.full((nc, D), C_init_scale, jnp.float32)

    for _ in range(3):
        l_tab = x_v.reshape(n_lits, D)
        lc_msgs = lc_k(l_tab, ck, lk, wb)
        c_arr = _cmlp(c_arr, lc_msgs, cw1, Cb1, CW2, Cb2)
        cl_part = cl_k(c_arr, ck, lk)
        mp_v = cl_part.reshape(NSC, nv, 2 * D)
        x_v = _lmlp(x_v, mp_v, lwa, lwb, lwc, Lb1, LW2, Lb2)

    out = _vmlp(x_v, VW1, Vb1, VW2, Vb2)
    return out[:, 0]
